# baseline (speedup 1.0000x reference)
import jax
import jax.numpy as jnp
from jax.experimental import pallas as pl
from jax.experimental.pallas import tpu as pltpu

B = 16384
S = 16
D = 128
BB = 64
BBS = BB * S


def _gru_block(st_ref, du_ref, wih_t_ref, whh_t_ref, b_ih_ref, b_hh_ref,
               ids_ref, out_ref):
    st = st_ref[...]
    smod = jax.lax.broadcasted_iota(jnp.int32, (BBS, 1), 0) % S
    sel = jnp.clip(ids_ref[...], 0, S - 1) == smod
    masked = st * sel.astype(jnp.float32)

    gid = jax.lax.broadcasted_iota(jnp.int32, (BB, BBS), 1) // S
    bid = jax.lax.broadcasted_iota(jnp.int32, (BB, BBS), 0)
    h_old = jnp.dot((gid == bid).astype(jnp.float32), masked,
                    preferred_element_type=jnp.float32)

    gi = jnp.dot(du_ref[...], wih_t_ref[...],
                 preferred_element_type=jnp.float32) + b_ih_ref[0]
    gh = jnp.dot(h_old, whh_t_ref[...],
                 preferred_element_type=jnp.float32) + b_hh_ref[0]
    r = jax.nn.sigmoid(gi[:, :D] + gh[:, :D])
    z = jax.nn.sigmoid(gi[:, D:2 * D] + gh[:, D:2 * D])
    n = jnp.tanh(gi[:, 2 * D:] + r * gh[:, 2 * D:])
    h_new = (1.0 - z) * n + z * h_old

    gid2 = jax.lax.broadcasted_iota(jnp.int32, (BBS, BB), 0) // S
    bid2 = jax.lax.broadcasted_iota(jnp.int32, (BBS, BB), 1)
    h_exp = jnp.dot((gid2 == bid2).astype(jnp.float32), h_new,
                    preferred_element_type=jnp.float32)
    out_ref[...] = jnp.where(sel, h_exp, st)


def kernel(states, delta_u, W_ih, W_hh, b_ih, b_hh, speaker_ids):
    nb = B // BB
    st2 = states.reshape(B * S, D)
    ids32 = speaker_ids.astype(jnp.int32)
    ids_exp = jnp.broadcast_to(ids32[:, None], (B, S)).reshape(B * S, 1)
    wih_t = W_ih.T
    whh_t = W_hh.T
    b_ih2 = b_ih.reshape(1, 3 * D)
    b_hh2 = b_hh.reshape(1, 3 * D)

    out = pl.pallas_call(
        _gru_block,
        grid=(nb,),
        in_specs=[
            pl.BlockSpec((BBS, D), lambda i: (i, 0)),
            pl.BlockSpec((BB, D), lambda i: (i, 0)),
            pl.BlockSpec((D, 3 * D), lambda i: (0, 0)),
            pl.BlockSpec((D, 3 * D), lambda i: (0, 0)),
            pl.BlockSpec((1, 3 * D), lambda i: (0, 0)),
            pl.BlockSpec((1, 3 * D), lambda i: (0, 0)),
            pl.BlockSpec((BBS, 1), lambda i: (i, 0)),
        ],
        out_specs=pl.BlockSpec((BBS, D), lambda i: (i, 0)),
        out_shape=jax.ShapeDtypeStruct((B * S, D), jnp.float32),
        compiler_params=pltpu.CompilerParams(
            dimension_semantics=("arbitrary",),
        ),
    )(st2, delta_u, wih_t, whh_t, b_ih2, b_hh2, ids_exp)
    return out.reshape(B, S, D)
